# initial kernel scaffold (unmeasured)
import functools

import jax
import jax.numpy as jnp
from jax import lax
from jax.experimental import pallas as pl
from jax.experimental.pallas import tpu as pltpu

N_DEV = 4


def _ag_body(x_ref, xg_ref, copy_sem, send_sems, recv_sems):
    my = lax.axis_index("i")
    left = lax.rem(my - 1 + N_DEV, N_DEV)
    right = lax.rem(my + 1, N_DEV)
    m = x_ref.shape[0]

    barrier_sem = pltpu.get_barrier_semaphore()
    for nbr in (left, right):
        pl.semaphore_signal(
            barrier_sem, inc=1, device_id=(nbr,),
            device_id_type=pl.DeviceIdType.MESH,
        )
    pl.semaphore_wait(barrier_sem, 2)

    cp = pltpu.make_async_copy(x_ref, xg_ref.at[pl.ds(my * m, m), :], copy_sem)
    cp.start()
    cp.wait()

    for h in range(N_DEV - 1):
        slot = lax.rem(my - h + N_DEV, N_DEV)
        rdma = pltpu.make_async_remote_copy(
            src_ref=xg_ref.at[pl.ds(slot * m, m), :],
            dst_ref=xg_ref.at[pl.ds(slot * m, m), :],
            send_sem=send_sems.at[h],
            recv_sem=recv_sems.at[h],
            device_id=(right,),
            device_id_type=pl.DeviceIdType.MESH,
        )
        rdma.start()
        rdma.wait()


def _all_gather(x):
    m, k = x.shape
    return pl.pallas_call(
        _ag_body,
        out_shape=jax.ShapeDtypeStruct((N_DEV * m, k), x.dtype),
        in_specs=[pl.BlockSpec(memory_space=pltpu.ANY)],
        out_specs=pl.BlockSpec(memory_space=pltpu.ANY),
        scratch_shapes=[
            pltpu.SemaphoreType.DMA,
            pltpu.SemaphoreType.DMA((N_DEV - 1,)),
            pltpu.SemaphoreType.DMA((N_DEV - 1,)),
        ],
        compiler_params=pltpu.CompilerParams(collective_id=0),
    )(x)


def _mm_body(xg_ref, w_ref, o_ref, acc_ref, *, k_steps):
    @pl.when(pl.program_id(1) == 0)
    def _():
        acc_ref[...] = jnp.zeros_like(acc_ref)

    acc_ref[...] += jnp.dot(
        xg_ref[...], w_ref[...], preferred_element_type=jnp.float32
    )

    @pl.when(pl.program_id(1) == k_steps - 1)
    def _():
        o_ref[...] = acc_ref[...]


def _matmul(xg, w):
    m_tot, k_tot = xg.shape
    _, n = w.shape
    bm, bk = 1024, 512
    k_steps = k_tot // bk
    return pl.pallas_call(
        functools.partial(_mm_body, k_steps=k_steps),
        grid=(m_tot // bm, k_steps),
        in_specs=[
            pl.BlockSpec((bm, bk), lambda i, k: (i, k)),
            pl.BlockSpec((bk, n), lambda i, k: (k, 0)),
        ],
        out_specs=pl.BlockSpec((bm, n), lambda i, k: (i, 0)),
        out_shape=jax.ShapeDtypeStruct((m_tot, n), jnp.float32),
        scratch_shapes=[pltpu.VMEM((bm, n), jnp.float32)],
        compiler_params=pltpu.CompilerParams(
            dimension_semantics=("parallel", "arbitrary"),
        ),
    )(xg, w)


def kernel(x, w_mat):
    xg = _all_gather(x)
    return _matmul(xg, w_mat)


# baseline (device time: 4453633 ns/iter reference)
import functools

import jax
import jax.numpy as jnp
from jax import lax
from jax.experimental import pallas as pl
from jax.experimental.pallas import tpu as pltpu

N_DEV = 4


def _ag_body(x_ref, xg_ref, copy_sem, send_sems, recv_sems):
    my = lax.axis_index("i")
    left = lax.rem(my - 1 + N_DEV, N_DEV)
    right = lax.rem(my + 1, N_DEV)
    m = x_ref.shape[0]

    barrier_sem = pltpu.get_barrier_semaphore()
    for nbr in (left, right):
        pl.semaphore_signal(
            barrier_sem, inc=1, device_id=(nbr,),
            device_id_type=pl.DeviceIdType.MESH,
        )
    pl.semaphore_wait(barrier_sem, 2)

    cp = pltpu.make_async_copy(x_ref, xg_ref.at[pl.ds(my * m, m), :], copy_sem)
    cp.start()
    cp.wait()

    for h in range(N_DEV - 1):
        slot = lax.rem(my - h + N_DEV, N_DEV)
        rdma = pltpu.make_async_remote_copy(
            src_ref=xg_ref.at[pl.ds(slot * m, m), :],
            dst_ref=xg_ref.at[pl.ds(slot * m, m), :],
            send_sem=send_sems.at[h],
            recv_sem=recv_sems.at[h],
            device_id=(right,),
            device_id_type=pl.DeviceIdType.MESH,
        )
        rdma.start()
        rdma.wait()


def _all_gather(x):
    m, k = x.shape
    return pl.pallas_call(
        _ag_body,
        out_shape=jax.ShapeDtypeStruct((N_DEV * m, k), x.dtype),
        in_specs=[pl.BlockSpec(memory_space=pltpu.MemorySpace.HBM)],
        out_specs=pl.BlockSpec(memory_space=pltpu.MemorySpace.HBM),
        scratch_shapes=[
            pltpu.SemaphoreType.DMA,
            pltpu.SemaphoreType.DMA((N_DEV - 1,)),
            pltpu.SemaphoreType.DMA((N_DEV - 1,)),
        ],
        compiler_params=pltpu.CompilerParams(collective_id=0),
    )(x)


def _mm_body(xg_ref, w_ref, o_ref, acc_ref, *, k_steps):
    @pl.when(pl.program_id(1) == 0)
    def _():
        acc_ref[...] = jnp.zeros_like(acc_ref)

    acc_ref[...] += jnp.dot(
        xg_ref[...], w_ref[...], preferred_element_type=jnp.float32
    )

    @pl.when(pl.program_id(1) == k_steps - 1)
    def _():
        o_ref[...] = acc_ref[...]


def _matmul(xg, w):
    m_tot, k_tot = xg.shape
    _, n = w.shape
    bm, bk = 1024, 512
    k_steps = k_tot // bk
    return pl.pallas_call(
        functools.partial(_mm_body, k_steps=k_steps),
        grid=(m_tot // bm, k_steps),
        in_specs=[
            pl.BlockSpec((bm, bk), lambda i, k: (i, k)),
            pl.BlockSpec((bk, n), lambda i, k: (k, 0)),
        ],
        out_specs=pl.BlockSpec((bm, n), lambda i, k: (i, 0)),
        out_shape=jax.ShapeDtypeStruct((m_tot, n), jnp.float32),
        scratch_shapes=[pltpu.VMEM((bm, n), jnp.float32)],
        compiler_params=pltpu.CompilerParams(
            dimension_semantics=("parallel", "arbitrary"),
        ),
    )(xg, w)


def kernel(x, w_mat):
    xg = _all_gather(x)
    return _matmul(xg, w_mat)
